# baseline (speedup 1.0000x reference)
import functools

import jax
import jax.numpy as jnp
from jax import lax
from jax.experimental import pallas as pl
from jax.experimental.pallas import tpu as pltpu
from jax.experimental.pallas import tpu_sc as plsc

N_NODES = 10000
N_EDGES = 320000
IN_DIM = 128
HID_DIM = 128
OUT_DIM = 64
HALF = HID_DIM // 2

NC, NS = 2, 16
NW = NC * NS
BLK = 128
NBUF = 5
LAG = 2
EP = 327680
EPT1 = EP // NS
NB1 = EPT1 // BLK
EPT2 = EP // NW
NB2 = EPT2 // BLK
NPAD = 10240
RPT = NPAD // NS
ROWB = 1024

_MESH = plsc.VectorSubcoreMesh(core_axis_name="c", subcore_axis_name="s",
                               num_cores=NC, num_subcores=NS)
_SC_PARAMS = pltpu.CompilerParams(use_tc_tiling_on_sc=False)


@functools.partial(
    pl.kernel,
    out_type=jax.ShapeDtypeStruct((NC * NPAD,), jnp.float32),
    mesh=_MESH,
    compiler_params=_SC_PARAMS,
    scratch_types=[
        pltpu.VMEM((NB2, BLK), jnp.int32),
        pltpu.VMEM((BLK,), jnp.float32),
        pltpu.VMEM((RPT,), jnp.float32),
        pltpu.VMEM_SHARED((NPAD,), jnp.float32),
    ],
)
def _deg_kernel(dst_hbm, out_hbm, dst_v, ones_v, buf_v, acc):
    c = lax.axis_index("c")
    s = lax.axis_index("s")
    w = c * NS + s

    for i in range(BLK // 16):
        ones_v[pl.ds(i * 16, 16)] = jnp.ones((16,), jnp.float32)

    def _zero(i, carry):
        buf_v[pl.ds(i * 16, 16)] = jnp.zeros((16,), jnp.float32)
        return carry
    lax.fori_loop(0, RPT // 16, _zero, 0)
    pltpu.sync_copy(buf_v, acc.at[pl.ds(s * RPT, RPT)])
    pltpu.sync_copy(dst_hbm.at[w], dst_v)
    plsc.subcore_barrier()

    def _block(j, carry):
        pltpu.sync_copy(ones_v, acc.at[dst_v.at[j]], add=True)
        return carry
    lax.fori_loop(0, NB2, _block, 0)
    plsc.subcore_barrier()

    pltpu.sync_copy(acc.at[pl.ds(s * RPT, RPT)], buf_v)
    pltpu.sync_copy(buf_v, out_hbm.at[pl.ds(c * NPAD + s * RPT, RPT)])


def _make_agg(D, ept, nb, feature_split):
    @functools.partial(
        pl.kernel,
        out_type=jax.ShapeDtypeStruct((NC * NPAD, D), jnp.float32),
        mesh=_MESH,
        compiler_params=_SC_PARAMS,
        scratch_types=[
            pltpu.VMEM((ept,), jnp.int32),
            pltpu.VMEM((nb, BLK), jnp.int32),
            *[pltpu.VMEM((BLK, D), jnp.float32) for _ in range(NBUF)],
            pltpu.VMEM_SHARED((NPAD, D), jnp.float32),
            pltpu.SemaphoreType.DMA,
            pltpu.SemaphoreType.DMA,
        ],
    )
    def _agg(g_hbm, src_hbm, dst_hbm, out_hbm,
             src_v, dst_v, *rest):
        ring = list(rest[:NBUF])
        acc, semg, sems = rest[NBUF:]
        buf_v = ring[0]
        c = lax.axis_index("c")
        s = lax.axis_index("s")
        slot = s if feature_split else c * NS + s

        def _zero(i, carry):
            for j in range(D // 16):
                buf_v[i, pl.ds(j * 16, 16)] = jnp.zeros((16,), jnp.float32)
            return carry
        lax.fori_loop(0, 128, _zero, 0)
        for k in range(RPT // 128):
            pltpu.sync_copy(buf_v, acc.at[pl.ds(s * RPT + k * 128, 128)])
        pltpu.sync_copy(src_hbm.at[slot], src_v)
        pltpu.sync_copy(dst_hbm.at[slot], dst_v)
        if feature_split:
            off = (c * NPAD).astype(jnp.int32)

            def _shift(i, carry):
                sl = pl.ds(i * 16, 16)
                src_v[sl] = src_v[sl] + off
                return carry
            lax.fori_loop(0, ept // 16, _shift, 0)
        plsc.subcore_barrier()

        def _gather_start(j, buf):
            pltpu.async_copy(g_hbm.at[src_v.at[pl.ds(j * BLK, BLK)]], buf, semg)

        def _gather_wait(j, buf):
            pltpu.make_async_copy(
                g_hbm.at[src_v.at[pl.ds(j * BLK, BLK)]], buf, semg).wait()

        def _scatter_start(j, buf):
            pltpu.async_copy(buf, acc.at[dst_v.at[j]], sems, add=True)

        def _scatter_wait(j, buf):
            pltpu.make_async_copy(buf, acc.at[dst_v.at[j]], sems).wait()

        for u in range(NBUF):
            _gather_start(u, ring[u])

        def _group(i, carry):
            for u in range(NBUF):
                j = NBUF * i + u
                _gather_wait(j, ring[u])
                _scatter_start(j, ring[u])
                up = (u - LAG) % NBUF
                jr = j - LAG + NBUF

                @pl.when(jnp.logical_and(j >= LAG, jr < nb))
                def _():
                    _scatter_wait(j - LAG, ring[up])
                    _gather_start(jr, ring[up])
            return carry
        lax.fori_loop(0, nb // NBUF, _group, 0)
        for u in range(NBUF):
            _scatter_wait(nb - NBUF + u, ring[u])
        plsc.subcore_barrier()

        for k in range(RPT // 128):
            r0 = s * RPT + k * 128
            pltpu.sync_copy(acc.at[pl.ds(r0, 128)], buf_v)
            pltpu.sync_copy(buf_v, out_hbm.at[pl.ds(c * NPAD + r0, 128)])

    return _agg


_agg1 = _make_agg(HALF, EPT1, NB1, feature_split=True)
_agg2 = _make_agg(OUT_DIM, EPT2, NB2, feature_split=False)


def _tc_first(x, W1, dinv):
    def body(x_ref, w_ref, d_ref, o_ref):
        t = jnp.dot(x_ref[...], w_ref[...],
                    preferred_element_type=jnp.float32) * d_ref[...]
        o_ref[0, :, :] = t[:, :HALF]
        o_ref[1, :, :] = t[:, HALF:]
    return pl.pallas_call(
        body,
        grid=(NPAD // ROWB,),
        in_specs=[
            pl.BlockSpec((ROWB, IN_DIM), lambda i: (i, 0)),
            pl.BlockSpec((IN_DIM, HID_DIM), lambda i: (0, 0)),
            pl.BlockSpec((ROWB, 1), lambda i: (i, 0)),
        ],
        out_specs=pl.BlockSpec((2, ROWB, HALF), lambda i: (0, i, 0)),
        out_shape=jax.ShapeDtypeStruct((2, NPAD, HALF), jnp.float32),
    )(x, W1, dinv)


def _tc_mid(p, g1, dinv, b1, W2):
    def body(p_ref, g_ref, d_ref, b_ref, w_ref, o_ref):
        d = d_ref[...]
        h0 = jnp.maximum((p_ref[0] + g_ref[0]) * d + b_ref[:, :HALF], 0.0)
        h1 = jnp.maximum((p_ref[1] + g_ref[1]) * d + b_ref[:, HALF:], 0.0)
        t = (jnp.dot(h0, w_ref[:HALF, :], preferred_element_type=jnp.float32)
             + jnp.dot(h1, w_ref[HALF:, :], preferred_element_type=jnp.float32))
        o_ref[...] = t * d
    return pl.pallas_call(
        body,
        grid=(NPAD // ROWB,),
        in_specs=[
            pl.BlockSpec((2, ROWB, HALF), lambda i: (0, i, 0)),
            pl.BlockSpec((2, ROWB, HALF), lambda i: (0, i, 0)),
            pl.BlockSpec((ROWB, 1), lambda i: (i, 0)),
            pl.BlockSpec((1, HID_DIM), lambda i: (0, 0)),
            pl.BlockSpec((HID_DIM, OUT_DIM), lambda i: (0, 0)),
        ],
        out_specs=pl.BlockSpec((ROWB, OUT_DIM), lambda i: (i, 0)),
        out_shape=jax.ShapeDtypeStruct((NPAD, OUT_DIM), jnp.float32),
    )(p, g1, dinv, b1, W2)


def _tc_last(p, g2, dinv, b2):
    def body(p_ref, g_ref, d_ref, b_ref, o_ref):
        y = (p_ref[0] + p_ref[1] + g_ref[...]) * d_ref[...] + b_ref[...]
        m = jnp.max(y, axis=1, keepdims=True)
        ex = jnp.exp(y - m)
        o_ref[...] = y - m - jnp.log(jnp.sum(ex, axis=1, keepdims=True))
    return pl.pallas_call(
        body,
        grid=(NPAD // ROWB,),
        in_specs=[
            pl.BlockSpec((2, ROWB, OUT_DIM), lambda i: (0, i, 0)),
            pl.BlockSpec((ROWB, OUT_DIM), lambda i: (i, 0)),
            pl.BlockSpec((ROWB, 1), lambda i: (i, 0)),
            pl.BlockSpec((1, OUT_DIM), lambda i: (0, 0)),
        ],
        out_specs=pl.BlockSpec((ROWB, OUT_DIM), lambda i: (i, 0)),
        out_shape=jax.ShapeDtypeStruct((NPAD, OUT_DIM), jnp.float32),
    )(p, g2, dinv, b2)


def kernel(x, edge_index, W1, b1, W2, b2):
    pad = EP - N_EDGES
    pad_ids = lax.iota(jnp.int32, pad)
    srcp = jnp.concatenate([edge_index[0], pad_ids % N_NODES])
    dstp = jnp.concatenate([edge_index[1],
                            N_NODES + pad_ids % (NPAD - N_NODES)])
    src1 = srcp.reshape(NS, EPT1)
    src2 = srcp.reshape(NW, EPT2)
    dst1 = dstp.reshape(NS, NB1, BLK)
    dst2 = dstp.reshape(NW, NB2, BLK)

    degp = _deg_kernel(dst2).reshape(NC, NPAD)
    dinv = lax.rsqrt(degp[0] + degp[1] + 1.0)[:, None]

    xp = jnp.zeros((NPAD, IN_DIM), jnp.float32).at[:N_NODES].set(x)
    g1 = _tc_first(xp, W1, dinv)
    p1 = _agg1(g1.reshape(NC * NPAD, HALF), src1, dst1).reshape(NC, NPAD, HALF)
    g2 = _tc_mid(p1, g1, dinv, b1[None, :], W2)
    p2 = _agg2(g2, src2, dst2).reshape(NC, NPAD, OUT_DIM)
    out = _tc_last(p2, g2, dinv, b2[None, :])
    return out[:N_NODES]


# speedup vs baseline: 1.0016x; 1.0016x over previous
import functools

import jax
import jax.numpy as jnp
import numpy as np
from jax import lax
from jax.experimental import pallas as pl
from jax.experimental.pallas import tpu as pltpu
from jax.experimental.pallas import tpu_sc as plsc

N_NODES = 10000
N_EDGES = 320000
IN_DIM = 128
HID_DIM = 128
OUT_DIM = 64
HALF = HID_DIM // 2

NC, NS = 2, 16
NW = NC * NS
BLK = 128
NBUF = 5
LAG = 2
EROWS = N_EDGES // BLK
NB1 = 160
NB2 = 80
PADROWS = 60
NPAD = 10240
RPT = NPAD // NS
ROWB = 1000

_MESH = plsc.VectorSubcoreMesh(core_axis_name="c", subcore_axis_name="s",
                               num_cores=NC, num_subcores=NS)
_SC_PARAMS = pltpu.CompilerParams(use_tc_tiling_on_sc=False)


def _load_blocks(hbm, pad_hbm, dst_ref, slot, nb):
    last = (EROWS // nb) * nb
    real = EROWS - last
    lastslot = EROWS // nb

    @pl.when(slot < lastslot)
    def _():
        pltpu.sync_copy(hbm.at[pl.ds(slot * nb, nb)], dst_ref)

    @pl.when(slot == lastslot)
    def _():
        pltpu.sync_copy(hbm.at[pl.ds(last, real)], dst_ref.at[pl.ds(0, real)])
        pltpu.sync_copy(pad_hbm, dst_ref.at[pl.ds(real, PADROWS)])


@functools.partial(
    pl.kernel,
    out_type=jax.ShapeDtypeStruct((NC * NPAD,), jnp.float32),
    mesh=_MESH,
    compiler_params=_SC_PARAMS,
    scratch_types=[
        pltpu.VMEM((NB2, BLK), jnp.int32),
        pltpu.VMEM((BLK,), jnp.float32),
        pltpu.VMEM((RPT,), jnp.float32),
        pltpu.VMEM_SHARED((NPAD,), jnp.float32),
    ],
)
def _deg_kernel(dst_hbm, pad_dst_hbm, out_hbm, dst_v, ones_v, buf_v, acc):
    c = lax.axis_index("c")
    s = lax.axis_index("s")
    w = c * NS + s

    for i in range(BLK // 16):
        ones_v[pl.ds(i * 16, 16)] = jnp.ones((16,), jnp.float32)

    def _zero(i, carry):
        buf_v[pl.ds(i * 16, 16)] = jnp.zeros((16,), jnp.float32)
        return carry
    lax.fori_loop(0, RPT // 16, _zero, 0)
    pltpu.sync_copy(buf_v, acc.at[pl.ds(s * RPT, RPT)])
    _load_blocks(dst_hbm, pad_dst_hbm, dst_v, w, NB2)
    plsc.subcore_barrier()

    def _block(j, carry):
        pltpu.sync_copy(ones_v, acc.at[dst_v.at[j]], add=True)
        return carry
    lax.fori_loop(0, NB2, _block, 0)
    plsc.subcore_barrier()

    pltpu.sync_copy(acc.at[pl.ds(s * RPT, RPT)], buf_v)
    pltpu.sync_copy(buf_v, out_hbm.at[pl.ds(c * NPAD + s * RPT, RPT)])


def _make_agg(D, nb, feature_split):
    @functools.partial(
        pl.kernel,
        out_type=jax.ShapeDtypeStruct((NC * NPAD, D), jnp.float32),
        mesh=_MESH,
        compiler_params=_SC_PARAMS,
        scratch_types=[
            pltpu.VMEM((nb, BLK), jnp.int32),
            pltpu.VMEM((nb, BLK), jnp.int32),
            *[pltpu.VMEM((BLK, D), jnp.float32) for _ in range(NBUF)],
            pltpu.VMEM_SHARED((NPAD, D), jnp.float32),
            pltpu.SemaphoreType.DMA,
            pltpu.SemaphoreType.DMA,
        ],
    )
    def _agg(g_hbm, src_hbm, dst_hbm, pad_src_hbm, pad_dst_hbm, out_hbm,
             src_v, dst_v, *rest):
        ring = list(rest[:NBUF])
        acc, semg, sems = rest[NBUF:]
        buf_v = ring[0]
        c = lax.axis_index("c")
        s = lax.axis_index("s")
        slot = s if feature_split else c * NS + s

        def _zero(i, carry):
            for j in range(D // 16):
                buf_v[i, pl.ds(j * 16, 16)] = jnp.zeros((16,), jnp.float32)
            return carry
        lax.fori_loop(0, 128, _zero, 0)
        for k in range(RPT // 128):
            pltpu.sync_copy(buf_v, acc.at[pl.ds(s * RPT + k * 128, 128)])
        _load_blocks(src_hbm, pad_src_hbm, src_v, slot, nb)
        _load_blocks(dst_hbm, pad_dst_hbm, dst_v, slot, nb)
        if feature_split:
            off = (c * N_NODES).astype(jnp.int32)

            def _shift(i, carry):
                for k in range(BLK // 16):
                    sl = pl.ds(k * 16, 16)
                    src_v[i, sl] = src_v[i, sl] + off
                return carry
            lax.fori_loop(0, nb, _shift, 0)
        plsc.subcore_barrier()

        def _gather_start(j, buf):
            pltpu.async_copy(g_hbm.at[src_v.at[j]], buf, semg)

        def _gather_wait(j, buf):
            pltpu.make_async_copy(g_hbm.at[src_v.at[j]], buf, semg).wait()

        def _scatter_start(j, buf):
            pltpu.async_copy(buf, acc.at[dst_v.at[j]], sems, add=True)

        def _scatter_wait(j, buf):
            pltpu.make_async_copy(buf, acc.at[dst_v.at[j]], sems).wait()

        for u in range(NBUF):
            _gather_start(u, ring[u])

        def _group(i, carry):
            for u in range(NBUF):
                j = NBUF * i + u
                _gather_wait(j, ring[u])
                _scatter_start(j, ring[u])
                up = (u - LAG) % NBUF
                jr = j - LAG + NBUF

                @pl.when(jnp.logical_and(j >= LAG, jr < nb))
                def _():
                    _scatter_wait(j - LAG, ring[up])
                    _gather_start(jr, ring[up])
            return carry
        lax.fori_loop(0, nb // NBUF, _group, 0)
        for u in range(NBUF):
            _scatter_wait(nb - NBUF + u, ring[u])
        plsc.subcore_barrier()

        for k in range(RPT // 128):
            r0 = s * RPT + k * 128
            pltpu.sync_copy(acc.at[pl.ds(r0, 128)], buf_v)
            pltpu.sync_copy(buf_v, out_hbm.at[pl.ds(c * NPAD + r0, 128)])

    return _agg


_agg1 = _make_agg(HALF, NB1, feature_split=True)
_agg2 = _make_agg(OUT_DIM, NB2, feature_split=False)


def _tc_first(x, W1, dinv):
    def body(x_ref, w_ref, d_ref, o_ref):
        t = jnp.dot(x_ref[...], w_ref[...],
                    preferred_element_type=jnp.float32) * d_ref[...]
        o_ref[0, :, :] = t[:, :HALF]
        o_ref[1, :, :] = t[:, HALF:]
    return pl.pallas_call(
        body,
        grid=(N_NODES // ROWB,),
        in_specs=[
            pl.BlockSpec((ROWB, IN_DIM), lambda i: (i, 0)),
            pl.BlockSpec((IN_DIM, HID_DIM), lambda i: (0, 0)),
            pl.BlockSpec((ROWB, 1), lambda i: (i, 0)),
        ],
        out_specs=pl.BlockSpec((2, ROWB, HALF), lambda i: (0, i, 0)),
        out_shape=jax.ShapeDtypeStruct((2, N_NODES, HALF), jnp.float32),
    )(x, W1, dinv)


def _tc_mid(p, g1, dinv, b1, W2):
    def body(p_ref, g_ref, d_ref, b_ref, w_ref, o_ref):
        d = d_ref[...]
        h0 = jnp.maximum((p_ref[0] + g_ref[0]) * d + b_ref[:, :HALF], 0.0)
        h1 = jnp.maximum((p_ref[1] + g_ref[1]) * d + b_ref[:, HALF:], 0.0)
        t = (jnp.dot(h0, w_ref[:HALF, :], preferred_element_type=jnp.float32)
             + jnp.dot(h1, w_ref[HALF:, :], preferred_element_type=jnp.float32))
        o_ref[...] = t * d
    return pl.pallas_call(
        body,
        grid=(N_NODES // ROWB,),
        in_specs=[
            pl.BlockSpec((2, ROWB, HALF), lambda i: (0, i, 0)),
            pl.BlockSpec((2, ROWB, HALF), lambda i: (0, i, 0)),
            pl.BlockSpec((ROWB, 1), lambda i: (i, 0)),
            pl.BlockSpec((1, HID_DIM), lambda i: (0, 0)),
            pl.BlockSpec((HID_DIM, OUT_DIM), lambda i: (0, 0)),
        ],
        out_specs=pl.BlockSpec((ROWB, OUT_DIM), lambda i: (i, 0)),
        out_shape=jax.ShapeDtypeStruct((N_NODES, OUT_DIM), jnp.float32),
    )(p, g1, dinv, b1, W2)


def _tc_last(p, g2, dinv, b2):
    def body(p_ref, g_ref, d_ref, b_ref, o_ref):
        y = (p_ref[0] + p_ref[1] + g_ref[...]) * d_ref[...] + b_ref[...]
        m = jnp.max(y, axis=1, keepdims=True)
        ex = jnp.exp(y - m)
        o_ref[...] = y - m - jnp.log(jnp.sum(ex, axis=1, keepdims=True))
    return pl.pallas_call(
        body,
        grid=(N_NODES // ROWB,),
        in_specs=[
            pl.BlockSpec((2, ROWB, OUT_DIM), lambda i: (0, i, 0)),
            pl.BlockSpec((ROWB, OUT_DIM), lambda i: (i, 0)),
            pl.BlockSpec((ROWB, 1), lambda i: (i, 0)),
            pl.BlockSpec((1, OUT_DIM), lambda i: (0, 0)),
        ],
        out_specs=pl.BlockSpec((ROWB, OUT_DIM), lambda i: (i, 0)),
        out_shape=jax.ShapeDtypeStruct((N_NODES, OUT_DIM), jnp.float32),
    )(p, g2, dinv, b2)


_ids = np.arange(PADROWS * BLK)
_PAD_SRC = np.asarray((_ids % N_NODES).reshape(PADROWS, BLK), np.int32)
_PAD_DST = np.asarray((N_NODES + _ids % (NPAD - N_NODES))
                      .reshape(PADROWS, BLK), np.int32)


def kernel(x, edge_index, W1, b1, W2, b2):
    src2d = edge_index[0].reshape(EROWS, BLK)
    dst2d = edge_index[1].reshape(EROWS, BLK)

    degp = _deg_kernel(dst2d, _PAD_DST).reshape(NC, NPAD)
    dinv = lax.rsqrt(degp[0] + degp[1] + 1.0)[:, None]

    g1 = _tc_first(x, W1, dinv[:N_NODES])
    p1 = _agg1(g1.reshape(NC * N_NODES, HALF),
               src2d, dst2d, _PAD_SRC, _PAD_DST).reshape(NC, NPAD, HALF)
    g2 = _tc_mid(p1, g1, dinv[:N_NODES], b1[None, :], W2)
    p2 = _agg2(g2, src2d, dst2d, _PAD_SRC, _PAD_DST).reshape(NC, NPAD, OUT_DIM)
    return _tc_last(p2, g2, dinv[:N_NODES], b2[None, :])


# speedup vs baseline: 1.0421x; 1.0404x over previous
import functools

import jax
import jax.numpy as jnp
import numpy as np
from jax import lax
from jax.experimental import pallas as pl
from jax.experimental.pallas import tpu as pltpu
from jax.experimental.pallas import tpu_sc as plsc

N_NODES = 10000
N_EDGES = 320000
IN_DIM = 128
HID_DIM = 128
OUT_DIM = 64
HALF = HID_DIM // 2

NC, NS = 2, 16
NW = NC * NS
BLK = 128
NBUF = 5
LAG = 2
EROWS = N_EDGES // BLK
NB1 = 160
NB2 = 80
PADROWS = 60
NPAD = 10240
RPT = NPAD // NS
ROWB = 1000

_MESH = plsc.VectorSubcoreMesh(core_axis_name="c", subcore_axis_name="s",
                               num_cores=NC, num_subcores=NS)
_SC_PARAMS = pltpu.CompilerParams(use_tc_tiling_on_sc=False)


def _load_blocks(hbm, pad_hbm, dst_ref, slot, nb):
    last = (EROWS // nb) * nb
    real = EROWS - last
    lastslot = EROWS // nb

    @pl.when(slot < lastslot)
    def _():
        pltpu.sync_copy(hbm.at[pl.ds(slot * nb, nb)], dst_ref)

    @pl.when(slot == lastslot)
    def _():
        pltpu.sync_copy(hbm.at[pl.ds(last, real)], dst_ref.at[pl.ds(0, real)])
        pltpu.sync_copy(pad_hbm, dst_ref.at[pl.ds(real, PADROWS)])


@functools.partial(
    pl.kernel,
    out_type=jax.ShapeDtypeStruct((NC * NPAD,), jnp.float32),
    mesh=_MESH,
    compiler_params=_SC_PARAMS,
    scratch_types=[
        pltpu.VMEM((NB2, BLK), jnp.int32),
        pltpu.VMEM((BLK,), jnp.float32),
        pltpu.VMEM((RPT,), jnp.float32),
        pltpu.VMEM_SHARED((NPAD,), jnp.float32),
    ],
)
def _deg_kernel(dst_hbm, pad_dst_hbm, out_hbm, dst_v, ones_v, buf_v, acc):
    c = lax.axis_index("c")
    s = lax.axis_index("s")
    w = c * NS + s

    for i in range(BLK // 16):
        ones_v[pl.ds(i * 16, 16)] = jnp.ones((16,), jnp.float32)

    def _zero(i, carry):
        buf_v[pl.ds(i * 16, 16)] = jnp.zeros((16,), jnp.float32)
        return carry
    lax.fori_loop(0, RPT // 16, _zero, 0)
    pltpu.sync_copy(buf_v, acc.at[pl.ds(s * RPT, RPT)])
    _load_blocks(dst_hbm, pad_dst_hbm, dst_v, w, NB2)
    plsc.subcore_barrier()

    def _block(j, carry):
        pltpu.sync_copy(ones_v, acc.at[dst_v.at[j]], add=True)
        return carry
    lax.fori_loop(0, NB2, _block, 0)
    plsc.subcore_barrier()

    pltpu.sync_copy(acc.at[pl.ds(s * RPT, RPT)], buf_v)
    pltpu.sync_copy(buf_v, out_hbm.at[pl.ds(c * NPAD + s * RPT, RPT)])


def _make_agg(D, nb, feature_split):
    @functools.partial(
        pl.kernel,
        out_type=jax.ShapeDtypeStruct((NC * NPAD, D), jnp.float32),
        mesh=_MESH,
        compiler_params=_SC_PARAMS,
        scratch_types=[
            pltpu.VMEM((nb, BLK), jnp.int32),
            pltpu.VMEM((nb, BLK), jnp.int32),
            *[pltpu.VMEM((BLK, D), jnp.float32) for _ in range(NBUF)],
            pltpu.VMEM_SHARED((NPAD, D), jnp.float32),
            pltpu.SemaphoreType.DMA,
            pltpu.SemaphoreType.DMA,
        ],
    )
    def _agg(g_hbm, src_hbm, dst_hbm, pad_src_hbm, pad_dst_hbm, out_hbm,
             src_v, dst_v, *rest):
        ring = list(rest[:NBUF])
        acc, semg, sems = rest[NBUF:]
        buf_v = ring[0]
        c = lax.axis_index("c")
        s = lax.axis_index("s")
        slot = s if feature_split else c * NS + s

        def _zero(i, carry):
            for j in range(D // 16):
                buf_v[i, pl.ds(j * 16, 16)] = jnp.zeros((16,), jnp.float32)
            return carry
        lax.fori_loop(0, 128, _zero, 0)
        for k in range(RPT // 128):
            pltpu.sync_copy(buf_v, acc.at[pl.ds(s * RPT + k * 128, 128)])
        _load_blocks(src_hbm, pad_src_hbm, src_v, slot, nb)
        _load_blocks(dst_hbm, pad_dst_hbm, dst_v, slot, nb)
        if feature_split:
            off = c.astype(jnp.int32)

            def _shift(i, carry):
                for k in range(BLK // 16):
                    sl = pl.ds(k * 16, 16)
                    v = src_v[i, sl]
                    src_v[i, sl] = v + v + off
                return carry
            lax.fori_loop(0, nb, _shift, 0)
        plsc.subcore_barrier()

        def _gather_start(j, buf):
            pltpu.async_copy(g_hbm.at[src_v.at[j]], buf, semg)

        def _gather_wait(j, buf):
            pltpu.make_async_copy(g_hbm.at[src_v.at[j]], buf, semg).wait()

        def _scatter_start(j, buf):
            pltpu.async_copy(buf, acc.at[dst_v.at[j]], sems, add=True)

        def _scatter_wait(j, buf):
            pltpu.make_async_copy(buf, acc.at[dst_v.at[j]], sems).wait()

        for u in range(NBUF):
            _gather_start(u, ring[u])

        def _group(i, carry):
            for u in range(NBUF):
                j = NBUF * i + u
                _gather_wait(j, ring[u])
                _scatter_start(j, ring[u])
                up = (u - LAG) % NBUF
                jr = j - LAG + NBUF

                @pl.when(jnp.logical_and(j >= LAG, jr < nb))
                def _():
                    _scatter_wait(j - LAG, ring[up])
                    _gather_start(jr, ring[up])
            return carry
        lax.fori_loop(0, nb // NBUF, _group, 0)
        for u in range(NBUF):
            _scatter_wait(nb - NBUF + u, ring[u])
        plsc.subcore_barrier()

        for k in range(RPT // 128):
            r0 = s * RPT + k * 128
            pltpu.sync_copy(acc.at[pl.ds(r0, 128)], buf_v)
            pltpu.sync_copy(buf_v, out_hbm.at[pl.ds(c * NPAD + r0, 128)])

    return _agg


_agg1 = _make_agg(HALF, NB1, feature_split=True)
_agg2 = _make_agg(OUT_DIM, NB2, feature_split=False)


def _tc_first(x, W1, dinv):
    def body(x_ref, w_ref, d_ref, o_ref):
        o_ref[...] = jnp.dot(x_ref[...], w_ref[...],
                             preferred_element_type=jnp.float32) * d_ref[...]
    return pl.pallas_call(
        body,
        grid=(N_NODES // ROWB,),
        in_specs=[
            pl.BlockSpec((ROWB, IN_DIM), lambda i: (i, 0)),
            pl.BlockSpec((IN_DIM, HID_DIM), lambda i: (0, 0)),
            pl.BlockSpec((ROWB, 1), lambda i: (i, 0)),
        ],
        out_specs=pl.BlockSpec((ROWB, HID_DIM), lambda i: (i, 0)),
        out_shape=jax.ShapeDtypeStruct((N_NODES, HID_DIM), jnp.float32),
    )(x, W1, dinv)


def _tc_mid(p, t1, dinv, b1, W2):
    def body(p_ref, t_ref, d_ref, b_ref, w_ref, o_ref):
        d = d_ref[...]
        h0 = jnp.maximum((p_ref[0] + t_ref[:, :HALF]) * d + b_ref[:, :HALF],
                         0.0)
        h1 = jnp.maximum((p_ref[1] + t_ref[:, HALF:]) * d + b_ref[:, HALF:],
                         0.0)
        t = (jnp.dot(h0, w_ref[:HALF, :], preferred_element_type=jnp.float32)
             + jnp.dot(h1, w_ref[HALF:, :], preferred_element_type=jnp.float32))
        o_ref[...] = t * d
    return pl.pallas_call(
        body,
        grid=(N_NODES // ROWB,),
        in_specs=[
            pl.BlockSpec((2, ROWB, HALF), lambda i: (0, i, 0)),
            pl.BlockSpec((ROWB, HID_DIM), lambda i: (i, 0)),
            pl.BlockSpec((ROWB, 1), lambda i: (i, 0)),
            pl.BlockSpec((1, HID_DIM), lambda i: (0, 0)),
            pl.BlockSpec((HID_DIM, OUT_DIM), lambda i: (0, 0)),
        ],
        out_specs=pl.BlockSpec((ROWB, OUT_DIM), lambda i: (i, 0)),
        out_shape=jax.ShapeDtypeStruct((N_NODES, OUT_DIM), jnp.float32),
    )(p, t1, dinv, b1, W2)


def _tc_last(p, g2, dinv, b2):
    def body(p_ref, g_ref, d_ref, b_ref, o_ref):
        y = (p_ref[0] + p_ref[1] + g_ref[...]) * d_ref[...] + b_ref[...]
        m = jnp.max(y, axis=1, keepdims=True)
        ex = jnp.exp(y - m)
        o_ref[...] = y - m - jnp.log(jnp.sum(ex, axis=1, keepdims=True))
    return pl.pallas_call(
        body,
        grid=(N_NODES // ROWB,),
        in_specs=[
            pl.BlockSpec((2, ROWB, OUT_DIM), lambda i: (0, i, 0)),
            pl.BlockSpec((ROWB, OUT_DIM), lambda i: (i, 0)),
            pl.BlockSpec((ROWB, 1), lambda i: (i, 0)),
            pl.BlockSpec((1, OUT_DIM), lambda i: (0, 0)),
        ],
        out_specs=pl.BlockSpec((ROWB, OUT_DIM), lambda i: (i, 0)),
        out_shape=jax.ShapeDtypeStruct((N_NODES, OUT_DIM), jnp.float32),
    )(p, g2, dinv, b2)


_ids = np.arange(PADROWS * BLK)
_PAD_SRC = np.asarray((_ids % N_NODES).reshape(PADROWS, BLK), np.int32)
_PAD_DST = np.asarray((N_NODES + _ids % (NPAD - N_NODES))
                      .reshape(PADROWS, BLK), np.int32)


def kernel(x, edge_index, W1, b1, W2, b2):
    src2d = edge_index[0].reshape(EROWS, BLK)
    dst2d = edge_index[1].reshape(EROWS, BLK)

    degp = _deg_kernel(dst2d, _PAD_DST).reshape(NC, NPAD)
    dinv = lax.rsqrt(degp[0] + degp[1] + 1.0)[:, None]

    t1 = _tc_first(x, W1, dinv[:N_NODES])
    p1 = _agg1(t1.reshape(NC * N_NODES, HALF),
               src2d, dst2d, _PAD_SRC, _PAD_DST).reshape(NC, NPAD, HALF)
    g2 = _tc_mid(p1, t1, dinv[:N_NODES], b1[None, :], W2)
    p2 = _agg2(g2, src2d, dst2d, _PAD_SRC, _PAD_DST).reshape(NC, NPAD, OUT_DIM)
    return _tc_last(p2, g2, dinv[:N_NODES], b2[None, :])
